# baseline (speedup 1.0000x reference)
import jax
import jax.numpy as jnp
from jax.experimental import pallas as pl
from jax.experimental.pallas import tpu as pltpu

N = 1048576
D = 32
K = 64
BLOCK = 4096


def _vq_block_kernel(x_ref, cbm2_ref, cbsq_ref, lt_ref, cb_ref, xq_ref, loss_ref):
    x = x_ref[...]
    mm = jax.lax.dot_general(x, cbm2_ref[...], (((1,), (1,)), ((), ())),
                             preferred_element_type=jnp.float32)
    s = mm + cbsq_ref[...]
    m = jnp.min(s, axis=1, keepdims=True)
    h = (s == m).astype(jnp.float32)
    hh = jax.lax.dot_general(h, lt_ref[...], (((1,), (0,)), ((), ())),
                             preferred_element_type=jnp.float32)
    onehot = jnp.where(hh == 0.0, h, 0.0)
    q = jax.lax.dot_general(onehot, cb_ref[...], (((1,), (0,)), ((), ())),
                            preferred_element_type=jnp.float32)
    r = q - x
    xq_ref[...] = x + r
    loss_ref[...] = jnp.full((1, 128), jnp.sum(r * r), jnp.float32)


def kernel(x, codebook):
    cbm2 = -2.0 * codebook
    cb_sq = jnp.sum(codebook * codebook, axis=1)[None, :]
    k_iota = jnp.arange(K, dtype=jnp.int32)
    lt = (k_iota[:, None] < k_iota[None, :]).astype(jnp.float32)
    grid = N // BLOCK
    x_q, loss_sum = pl.pallas_call(
        _vq_block_kernel,
        grid=(grid,),
        in_specs=[
            pl.BlockSpec((BLOCK, D), lambda i: (i, 0)),
            pl.BlockSpec((K, D), lambda i: (0, 0)),
            pl.BlockSpec((1, K), lambda i: (0, 0)),
            pl.BlockSpec((K, K), lambda i: (0, 0)),
            pl.BlockSpec((K, D), lambda i: (0, 0)),
        ],
        out_specs=[
            pl.BlockSpec((BLOCK, D), lambda i: (i, 0)),
            pl.BlockSpec((1, 128), lambda i: (0, i)),
        ],
        out_shape=[
            jax.ShapeDtypeStruct((N, D), jnp.float32),
            jax.ShapeDtypeStruct((1, grid * 128), jnp.float32),
        ],
        compiler_params=pltpu.CompilerParams(
            dimension_semantics=("parallel",),
        ),
    )(x, cbm2, cb_sq, lt, codebook)
    l_vq = (jnp.sum(loss_sum.reshape(grid, 128)[:, 0]) / (N * D)).reshape(())
    return (x_q, l_vq)


# speedup vs baseline: 1.1530x; 1.1530x over previous
import jax
import jax.numpy as jnp
from jax.experimental import pallas as pl
from jax.experimental.pallas import tpu as pltpu

N = 1048576
D = 32
K = 64
BLOCK = 16384


def _vq_block_kernel(x_ref, cbm2_ref, cbsq_ref, lt_ref, cb_ref, xq_ref, loss_ref):
    x = x_ref[...]
    mm = jax.lax.dot_general(x, cbm2_ref[...], (((1,), (1,)), ((), ())),
                             preferred_element_type=jnp.float32)
    s = mm + cbsq_ref[...]
    m = jnp.min(s, axis=1, keepdims=True)
    h = (s == m).astype(jnp.float32)
    hh = jax.lax.dot_general(h, lt_ref[...], (((1,), (0,)), ((), ())),
                             preferred_element_type=jnp.float32)
    onehot = jnp.where(hh == 0.0, h, 0.0)
    q = jax.lax.dot_general(onehot, cb_ref[...], (((1,), (0,)), ((), ())),
                            preferred_element_type=jnp.float32)
    r = q - x
    xq_ref[...] = x + r
    loss_ref[...] = jnp.full((1, 128), jnp.sum(r * r), jnp.float32)


def kernel(x, codebook):
    cbm2 = -2.0 * codebook
    cb_sq = jnp.sum(codebook * codebook, axis=1)[None, :]
    k_iota = jnp.arange(K, dtype=jnp.int32)
    lt = (k_iota[:, None] < k_iota[None, :]).astype(jnp.float32)
    grid = N // BLOCK
    x_q, loss_sum = pl.pallas_call(
        _vq_block_kernel,
        grid=(grid,),
        in_specs=[
            pl.BlockSpec((BLOCK, D), lambda i: (i, 0)),
            pl.BlockSpec((K, D), lambda i: (0, 0)),
            pl.BlockSpec((1, K), lambda i: (0, 0)),
            pl.BlockSpec((K, K), lambda i: (0, 0)),
            pl.BlockSpec((K, D), lambda i: (0, 0)),
        ],
        out_specs=[
            pl.BlockSpec((BLOCK, D), lambda i: (i, 0)),
            pl.BlockSpec((1, 128), lambda i: (0, i)),
        ],
        out_shape=[
            jax.ShapeDtypeStruct((N, D), jnp.float32),
            jax.ShapeDtypeStruct((1, grid * 128), jnp.float32),
        ],
        compiler_params=pltpu.CompilerParams(
            dimension_semantics=("parallel",),
        ),
    )(x, cbm2, cb_sq, lt, codebook)
    l_vq = (jnp.sum(loss_sum.reshape(grid, 128)[:, 0]) / (N * D)).reshape(())
    return (x_q, l_vq)


# speedup vs baseline: 1.1560x; 1.0026x over previous
import jax
import jax.numpy as jnp
from jax.experimental import pallas as pl
from jax.experimental.pallas import tpu as pltpu

N = 1048576
D = 32
K = 64
BLOCK = 16384


def _vq_block_kernel(x_ref, cbm2_ref, cbsq_ref, lt_ref, cb_ref, xq_ref, loss_ref):
    x = x_ref[...]
    mm = jax.lax.dot_general(x, cbm2_ref[...], (((1,), (1,)), ((), ())),
                             preferred_element_type=jnp.float32)
    s = mm + cbsq_ref[...]
    m = jnp.min(s, axis=1, keepdims=True)
    h = (s == m).astype(jnp.float32)
    hh = jax.lax.dot_general(h, lt_ref[...], (((1,), (0,)), ((), ())),
                             preferred_element_type=jnp.float32)
    onehot = jnp.where(hh == 0.0, h, 0.0)
    q = jax.lax.dot_general(onehot, cb_ref[...], (((1,), (0,)), ((), ())),
                            preferred_element_type=jnp.float32)
    xq_ref[...] = q
    psum = jnp.sum(m) + jnp.sum(x * x)
    loss_ref[...] = jnp.full((1, 128), psum, jnp.float32)


def kernel(x, codebook):
    cbm2 = -2.0 * codebook
    cb_sq = jnp.sum(codebook * codebook, axis=1)[None, :]
    k_iota = jnp.arange(K, dtype=jnp.int32)
    lt = (k_iota[:, None] < k_iota[None, :]).astype(jnp.float32)
    grid = N // BLOCK
    x_q, loss_sum = pl.pallas_call(
        _vq_block_kernel,
        grid=(grid,),
        in_specs=[
            pl.BlockSpec((BLOCK, D), lambda i: (i, 0)),
            pl.BlockSpec((K, D), lambda i: (0, 0)),
            pl.BlockSpec((1, K), lambda i: (0, 0)),
            pl.BlockSpec((K, K), lambda i: (0, 0)),
            pl.BlockSpec((K, D), lambda i: (0, 0)),
        ],
        out_specs=[
            pl.BlockSpec((BLOCK, D), lambda i: (i, 0)),
            pl.BlockSpec((1, 128), lambda i: (0, i)),
        ],
        out_shape=[
            jax.ShapeDtypeStruct((N, D), jnp.float32),
            jax.ShapeDtypeStruct((1, grid * 128), jnp.float32),
        ],
        compiler_params=pltpu.CompilerParams(
            dimension_semantics=("parallel",),
        ),
    )(x, cbm2, cb_sq, lt, codebook)
    l_vq = (jnp.sum(loss_sum.reshape(grid, 128)[:, 0]) / (N * D)).reshape(())
    return (x_q, l_vq)
